# baseline (speedup 1.0000x reference)
import functools

import jax
import jax.numpy as jnp
from jax import lax
from jax.experimental import pallas as pl
from jax.experimental.pallas import tpu as pltpu
from jax.experimental.pallas import tpu_sc as plsc

B = 16384
D = 64
NC = 2
NS = 16
NW = NC * NS
BPW = B // NW
IDX_CH = BPW // 128

BT = 2048
NB = B // BT

_mesh = plsc.VectorSubcoreMesh(core_axis_name="c", subcore_axis_name="s")


@functools.partial(
    pl.kernel,
    mesh=_mesh,
    compiler_params=pltpu.CompilerParams(use_tc_tiling_on_sc=False),
    out_type=[
        jax.ShapeDtypeStruct((B, D), jnp.float32),
        jax.ShapeDtypeStruct((B, D), jnp.float32),
    ],
    scratch_types=[
        pltpu.VMEM((IDX_CH, 128), jnp.int32),
        pltpu.VMEM((IDX_CH, 128), jnp.int32),
        pltpu.VMEM((BPW, D), jnp.float32),
        pltpu.VMEM((BPW, D), jnp.float32),
        pltpu.SemaphoreType.DMA,
        pltpu.SemaphoreType.DMA,
    ],
)
def _sc_gather(uidx_hbm, midx_hbm, uf_hbm, mf_hbm, ue_out, me_out,
               uidx_v, midx_v, urows_v, mrows_v, sem_u, sem_m):
    wid = lax.axis_index("s") * NC + lax.axis_index("c")
    base = wid * BPW
    pltpu.sync_copy(uidx_hbm.at[wid], uidx_v)
    pltpu.sync_copy(midx_hbm.at[wid], midx_v)
    u_copies = []
    m_copies = []
    for j in range(IDX_CH):
        u_copies.append(pltpu.async_copy(
            uf_hbm.at[uidx_v.at[j]], urows_v.at[pl.ds(j * 128, 128)], sem_u))
        m_copies.append(pltpu.async_copy(
            mf_hbm.at[midx_v.at[j]], mrows_v.at[pl.ds(j * 128, 128)], sem_m))
    for c in u_copies:
        c.wait()
    pltpu.sync_copy(urows_v, ue_out.at[pl.ds(base, BPW)])
    for c in m_copies:
        c.wait()
    pltpu.sync_copy(mrows_v, me_out.at[pl.ds(base, BPW)])


def _tc_mlp_body(ue_ref, me_ref, g_ref, gf_ref, w1_ref, b1_ref, w2_ref, b2_ref,
                 w3_ref, b3_ref, out_ref):
    ue = ue_ref[...]
    me = me_ref[...]
    g = g_ref[0]
    ohT = (lax.broadcasted_iota(jnp.int32, (32, BT), 0) == g).astype(jnp.float32)
    gf = gf_ref[...]
    w1 = w1_ref[...]
    h1 = jnp.dot(ue, w1[0:D, :], preferred_element_type=jnp.float32)
    h1 += jnp.dot(me, w1[D:2 * D, :], preferred_element_type=jnp.float32)
    gcon = jnp.dot(gf, w1[2 * D:3 * D, :], preferred_element_type=jnp.float32)
    h1 += lax.dot_general(ohT, gcon, (((0,), (0,)), ((), ())),
                          preferred_element_type=jnp.float32)
    h1 = jnp.maximum(h1 + b1_ref[...], 0.0)
    h2 = jnp.dot(h1, w2_ref[...], preferred_element_type=jnp.float32)
    h2 = jnp.maximum(h2 + b2_ref[...], 0.0)
    out = jnp.sum(h2 * w3_ref[...], axis=1, keepdims=True) + b3_ref[...]
    out_ref[...] = out


_tc_mlp = pl.pallas_call(
    _tc_mlp_body,
    grid=(NB,),
    in_specs=[
        pl.BlockSpec((BT, D), lambda i: (i, 0)),
        pl.BlockSpec((BT, D), lambda i: (i, 0)),
        pl.BlockSpec((1, 1, BT), lambda i: (i, 0, 0)),
        pl.BlockSpec((32, D), lambda i: (0, 0)),
        pl.BlockSpec((3 * D, 128), lambda i: (0, 0)),
        pl.BlockSpec((1, 128), lambda i: (0, 0)),
        pl.BlockSpec((128, D), lambda i: (0, 0)),
        pl.BlockSpec((1, D), lambda i: (0, 0)),
        pl.BlockSpec((1, D), lambda i: (0, 0)),
        pl.BlockSpec((1, 1), lambda i: (0, 0)),
    ],
    out_specs=pl.BlockSpec((BT, 1), lambda i: (i, 0)),
    out_shape=jax.ShapeDtypeStruct((B, 1), jnp.float32),
)


def kernel(user, movie, genres, user_factors, movie_factors, genre_factors,
           W1, b1, W2, b2, W3, b3):
    uidx = user.astype(jnp.int32).reshape(NW, IDX_CH, 128)
    midx = movie.astype(jnp.int32).reshape(NW, IDX_CH, 128)
    ue, me = _sc_gather(uidx, midx, user_factors, movie_factors)
    g3 = genres.astype(jnp.int32).reshape(NB, 1, BT)
    return _tc_mlp(ue, me, g3, genre_factors,
                   W1, b1.reshape(1, 128), W2, b2.reshape(1, D),
                   W3.reshape(1, D), b3.reshape(1, 1))


# speedup vs baseline: 1.5514x; 1.5514x over previous
import functools

import jax
import jax.numpy as jnp
from jax import lax
from jax.experimental import pallas as pl
from jax.experimental.pallas import tpu as pltpu
from jax.experimental.pallas import tpu_sc as plsc

B = 16384
D = 64
NC = 2
NS = 16
NW = NC * NS
BPW = B // NW
G = 64

BT = 2048
NB = B // BT

_mesh = plsc.VectorSubcoreMesh(core_axis_name="c", subcore_axis_name="s")


def _gather_rows(wid, idx_hbm, table, out_hbm, idx_cv, rowbuf, sem):
    def chunk_body(ch, carry):
        pltpu.sync_copy(idx_hbm.at[wid, ch], idx_cv)
        for g4 in range(G // 16):
            raw = idx_cv[pl.ds(g4 * 16, 16)]
            for l in range(16):
                r = raw[l]
                pltpu.async_copy(table.at[r], rowbuf.at[g4 * 16 + l], sem)
        pltpu.make_async_copy(table.at[pl.ds(0, G)], rowbuf, sem).wait()
        pltpu.sync_copy(rowbuf, out_hbm.at[pl.ds(wid * BPW + ch * G, G)])
        return carry

    lax.fori_loop(0, BPW // G, chunk_body, 0)


@functools.partial(
    pl.kernel,
    mesh=_mesh,
    out_type=[
        jax.ShapeDtypeStruct((B, D), jnp.float32),
        jax.ShapeDtypeStruct((B, D), jnp.float32),
    ],
    scratch_types=[
        pltpu.VMEM((G,), jnp.int32),
        pltpu.VMEM((G, D), jnp.float32),
        pltpu.SemaphoreType.DMA,
    ],
)
def _sc_gather(uidx_hbm, midx_hbm, uf_hbm, mf_hbm, ue_out, me_out,
               idx_cv, rowbuf, sem):
    wid = lax.axis_index("s") * NC + lax.axis_index("c")
    _gather_rows(wid, uidx_hbm, uf_hbm, ue_out, idx_cv, rowbuf, sem)
    _gather_rows(wid, midx_hbm, mf_hbm, me_out, idx_cv, rowbuf, sem)


def _tc_mlp_body(ue_ref, me_ref, g_ref, gf_ref, w1_ref, b1_ref, w2_ref, b2_ref,
                 w3_ref, b3_ref, out_ref):
    ue = ue_ref[...]
    me = me_ref[...]
    g = g_ref[0]
    ohT = (lax.broadcasted_iota(jnp.int32, (32, BT), 0) == g).astype(jnp.float32)
    gf = gf_ref[...]
    w1 = w1_ref[...]
    h1 = jnp.dot(ue, w1[0:D, :], preferred_element_type=jnp.float32)
    h1 += jnp.dot(me, w1[D:2 * D, :], preferred_element_type=jnp.float32)
    gcon = jnp.dot(gf, w1[2 * D:3 * D, :], preferred_element_type=jnp.float32)
    h1 += lax.dot_general(ohT, gcon, (((0,), (0,)), ((), ())),
                          preferred_element_type=jnp.float32)
    h1 = jnp.maximum(h1 + b1_ref[...], 0.0)
    h2 = jnp.dot(h1, w2_ref[...], preferred_element_type=jnp.float32)
    h2 = jnp.maximum(h2 + b2_ref[...], 0.0)
    out = jnp.sum(h2 * w3_ref[...], axis=1, keepdims=True) + b3_ref[...]
    out_ref[...] = out


_tc_mlp = pl.pallas_call(
    _tc_mlp_body,
    grid=(NB,),
    in_specs=[
        pl.BlockSpec((BT, D), lambda i: (i, 0)),
        pl.BlockSpec((BT, D), lambda i: (i, 0)),
        pl.BlockSpec((1, 1, BT), lambda i: (i, 0, 0)),
        pl.BlockSpec((32, D), lambda i: (0, 0)),
        pl.BlockSpec((3 * D, 128), lambda i: (0, 0)),
        pl.BlockSpec((1, 128), lambda i: (0, 0)),
        pl.BlockSpec((128, D), lambda i: (0, 0)),
        pl.BlockSpec((1, D), lambda i: (0, 0)),
        pl.BlockSpec((1, D), lambda i: (0, 0)),
        pl.BlockSpec((1, 1), lambda i: (0, 0)),
    ],
    out_specs=pl.BlockSpec((BT, 1), lambda i: (i, 0)),
    out_shape=jax.ShapeDtypeStruct((B, 1), jnp.float32),
)


def kernel(user, movie, genres, user_factors, movie_factors, genre_factors,
           W1, b1, W2, b2, W3, b3):
    uidx = user.astype(jnp.int32).reshape(NW, BPW // G, G)
    midx = movie.astype(jnp.int32).reshape(NW, BPW // G, G)
    ue, me = _sc_gather(uidx, midx, user_factors, movie_factors)
    g3 = genres.astype(jnp.int32).reshape(NB, 1, BT)
    return _tc_mlp(ue, me, g3, genre_factors,
                   W1, b1.reshape(1, 128), W2, b2.reshape(1, D),
                   W3.reshape(1, D), b3.reshape(1, 1))
